# Initial kernel scaffold; baseline (speedup 1.0000x reference)
import jax
import jax.numpy as jnp
from jax.experimental import pallas as pl


def kernel(x, edge_index, edgenet_input, pae_w1, pae_b1, pae_w2, pae_b2, conv1_w, conv1_b, conv2_w, conv2_b, lin1_w, lin1_b, lin2_w, lin2_b):
    raise NotImplementedError("write your pallas kernel here")


# speedup vs baseline: 10.6178x; 10.6178x over previous
import functools

import jax
import jax.numpy as jnp
from jax import lax
from jax.experimental import pallas as pl
from jax.experimental.pallas import tpu as pltpu
from jax.experimental.pallas import tpu_sc as plsc

NC = 2
NS = 16
NW = NC * NS
LANES = 16
K = 128


def _mlp_body(en_ref, w1t_ref, b1c_ref, w2t_ref, b2_ref, out_ref):
    en = en_ref[0]
    h = jnp.dot(w1t_ref[...], en, preferred_element_type=jnp.float32)
    h = jnp.maximum(h + b1c_ref[...], 0.0)
    s = jnp.dot(w2t_ref[...], h, preferred_element_type=jnp.float32)
    out_ref[0] = jax.nn.sigmoid(s + b2_ref[...])


def _edge_mlp(en_t, w1, b1, w2, b2):
    E = en_t.shape[1]
    RB = 8000
    grid = E // RB
    en3 = en_t.reshape(2, grid, RB).swapaxes(0, 1)
    out = pl.pallas_call(
        _mlp_body,
        grid=(grid,),
        in_specs=[
            pl.BlockSpec((1, 2, RB), lambda i: (i, 0, 0)),
            pl.BlockSpec((64, 2), lambda i: (0, 0)),
            pl.BlockSpec((64, 1), lambda i: (0, 0)),
            pl.BlockSpec((1, 64), lambda i: (0, 0)),
            pl.BlockSpec((1, 1), lambda i: (0, 0)),
        ],
        out_specs=pl.BlockSpec((1, 1, RB), lambda i: (i, 0, 0)),
        out_shape=jax.ShapeDtypeStruct((grid, 1, RB), jnp.float32),
    )(en3, w1.T, b1[:, None], w2.T, b2[None, :])
    return out.reshape(E)


def _mm_body(x_ref, w_ref, o_ref):
    o_ref[...] = jnp.dot(x_ref[...], w_ref[...],
                         preferred_element_type=jnp.float32)


def _matmul(x, w):
    N, Kd = x.shape
    M = w.shape[1]
    R = 2000
    return pl.pallas_call(
        _mm_body,
        grid=(N // R,),
        in_specs=[
            pl.BlockSpec((R, Kd), lambda i: (i, 0)),
            pl.BlockSpec((Kd, M), lambda i: (0, 0)),
        ],
        out_specs=pl.BlockSpec((R, M), lambda i: (i, 0)),
        out_shape=jax.ShapeDtypeStruct((N, M), jnp.float32),
    )(x, w)


def _dinv_of(p):
    return lax.rsqrt(1.0 + p[:, 0:1] + p[:, 1:2])


def _prep_body(degt_ref, xw_ref, y_ref):
    y_ref[...] = xw_ref[...] * _dinv_of(degt_ref[...])


def _prep(degt, xw):
    N, D = xw.shape
    R = 2000
    return pl.pallas_call(
        _prep_body,
        grid=(N // R,),
        in_specs=[
            pl.BlockSpec((R, 2), lambda i: (i, 0)),
            pl.BlockSpec((R, D), lambda i: (i, 0)),
        ],
        out_specs=pl.BlockSpec((R, D), lambda i: (i, 0)),
        out_shape=jax.ShapeDtypeStruct((N, D), jnp.float32),
    )(degt, xw)


def _mid_body(acc_ref, y_ref, degt_ref, b_ref, w2_ref, y2_ref):
    dinv = _dinv_of(degt_ref[...])
    t = (acc_ref[0] + acc_ref[1] + y_ref[...]) * dinv + b_ref[...]
    h1 = jnp.maximum(t, 0.0)
    y2_ref[...] = jnp.dot(h1, w2_ref[...],
                          preferred_element_type=jnp.float32) * dinv


def _mid(acc, y, degt, b, w2):
    N, D = y.shape
    R = 2000
    return pl.pallas_call(
        _mid_body,
        grid=(N // R,),
        in_specs=[
            pl.BlockSpec((2, R, D), lambda i: (0, i, 0)),
            pl.BlockSpec((R, D), lambda i: (i, 0)),
            pl.BlockSpec((R, 2), lambda i: (i, 0)),
            pl.BlockSpec((1, D), lambda i: (0, 0)),
            pl.BlockSpec((D, D), lambda i: (0, 0)),
        ],
        out_specs=pl.BlockSpec((R, D), lambda i: (i, 0)),
        out_shape=jax.ShapeDtypeStruct((N, D), jnp.float32),
    )(acc, y, degt, b[None, :], w2)


def _fin_body(acc_ref, y_ref, degt_ref, b_ref, l1w_ref, l1b_ref,
              l2w_ref, l2b_ref, o_ref):
    dinv = _dinv_of(degt_ref[...])
    t = (acc_ref[0] + acc_ref[1] + y_ref[...]) * dinv + b_ref[...]
    h2 = jnp.maximum(t, 0.0)
    h3 = jnp.dot(h2, l1w_ref[...], preferred_element_type=jnp.float32)
    h3 = jnp.maximum(h3 + l1b_ref[...], 0.0)
    o_ref[...] = jnp.dot(h3, l2w_ref[...],
                         preferred_element_type=jnp.float32) + l2b_ref[...]


def _final(acc, y, degt, b, l1w, l1b, l2wp, l2bp):
    N, D = y.shape
    H = l1w.shape[1]
    R = 2000
    return pl.pallas_call(
        _fin_body,
        grid=(N // R,),
        in_specs=[
            pl.BlockSpec((2, R, D), lambda i: (0, i, 0)),
            pl.BlockSpec((R, D), lambda i: (i, 0)),
            pl.BlockSpec((R, 2), lambda i: (i, 0)),
            pl.BlockSpec((1, D), lambda i: (0, 0)),
            pl.BlockSpec((D, H), lambda i: (0, 0)),
            pl.BlockSpec((1, H), lambda i: (0, 0)),
            pl.BlockSpec((H, D), lambda i: (0, 0)),
            pl.BlockSpec((1, D), lambda i: (0, 0)),
        ],
        out_specs=pl.BlockSpec((R, D), lambda i: (i, 0)),
        out_shape=jax.ShapeDtypeStruct((N, D), jnp.float32),
    )(acc, y, degt, b[None, :], l1w, l1b[None, :], l2wp, l2bp[None, :])


def _worker_chunks(E):
    nch_total = E // K
    base = nch_total // NW
    rem = nch_total % NW
    return base, rem


def _sc_degree(dst, ew, N):
    E = dst.shape[0]
    base_nch, rem = _worker_chunks(E)
    ZC = 2000

    mesh = plsc.VectorSubcoreMesh(core_axis_name="c", subcore_axis_name="s")

    @functools.partial(
        pl.kernel,
        out_type=jax.ShapeDtypeStruct((NC, N), jnp.float32),
        mesh=mesh,
        compiler_params=pltpu.CompilerParams(needs_layout_passes=False),
        scratch_types=[
            pltpu.VMEM((K,), jnp.int32),
            pltpu.VMEM((K,), jnp.float32),
            pltpu.VMEM((ZC,), jnp.float32),
            pltpu.VMEM_SHARED((N,), jnp.float32),
        ],
    )
    def k(dst_hbm, ew_hbm, out_hbm, dst_v, ew_v, zbuf, deg_sh):
        cid = lax.axis_index("c")
        sid = lax.axis_index("s")
        wid = sid * NC + cid

        def zb(i, _):
            zbuf[pl.ds(i * LANES, LANES)] = jnp.zeros((LANES,), jnp.float32)
            return _
        lax.fori_loop(0, ZC // LANES, zb, None)

        @pl.when(sid == 0)
        def _():
            for i in range(N // ZC):
                pltpu.sync_copy(zbuf, deg_sh.at[pl.ds(i * ZC, ZC)])

        plsc.subcore_barrier()

        nch = base_nch + jnp.where(wid < rem, 1, 0)

        def body(i, _):
            bse = (wid + i * NW) * K
            pltpu.sync_copy(dst_hbm.at[pl.ds(bse, K)], dst_v)
            pltpu.sync_copy(ew_hbm.at[pl.ds(bse, K)], ew_v)
            pltpu.sync_copy(ew_v, deg_sh.at[dst_v], add=True)
            return _
        lax.fori_loop(0, nch, body, None)

        plsc.subcore_barrier()

        @pl.when(sid == 0)
        def _():
            pltpu.sync_copy(deg_sh, out_hbm.at[cid])

    return k(dst, ew)


def _sc_messages(y, src, dst, ew):
    N, D = y.shape
    E = src.shape[0]
    base_nch, rem = _worker_chunks(E)
    ZR = 80
    NBLK = N // ZR

    mesh = plsc.VectorSubcoreMesh(core_axis_name="c", subcore_axis_name="s")

    @functools.partial(
        pl.kernel,
        out_type=jax.ShapeDtypeStruct((NC, N, D), jnp.float32),
        mesh=mesh,
        compiler_params=pltpu.CompilerParams(needs_layout_passes=False),
        scratch_types=[
            pltpu.VMEM((K,), jnp.int32),
            pltpu.VMEM((K,), jnp.int32),
            pltpu.VMEM((K,), jnp.float32),
            pltpu.VMEM((K, D), jnp.float32),
            pltpu.VMEM((ZR, D), jnp.float32),
            pltpu.VMEM_SHARED((N, D), jnp.float32),
        ],
    )
    def k(y_hbm, src_hbm, dst_hbm, ew_hbm, out_hbm,
          src_v, dst_v, ew_v, rows, zbuf, acc_sh):
        cid = lax.axis_index("c")
        sid = lax.axis_index("s")
        wid = sid * NC + cid

        def zb(r, _):
            for c in range(D // LANES):
                zbuf[r, pl.ds(c * LANES, LANES)] = jnp.zeros(
                    (LANES,), jnp.float32)
            return _
        lax.fori_loop(0, ZR, zb, None)

        nblk_s = NBLK // NS + jnp.where(sid < NBLK % NS, 1, 0)

        def zc(i, _):
            b = sid + i * NS
            pltpu.sync_copy(zbuf, acc_sh.at[pl.ds(b * ZR, ZR)])
            return _
        lax.fori_loop(0, nblk_s, zc, None)

        plsc.subcore_barrier()

        nch = base_nch + jnp.where(wid < rem, 1, 0)

        def body(i, _):
            bse = (wid + i * NW) * K
            pltpu.sync_copy(src_hbm.at[pl.ds(bse, K)], src_v)
            pltpu.sync_copy(dst_hbm.at[pl.ds(bse, K)], dst_v)
            pltpu.sync_copy(ew_hbm.at[pl.ds(bse, K)], ew_v)
            pltpu.sync_copy(y_hbm.at[src_v], rows)

            def scale(e, _):
                s = plsc.load_gather(
                    ew_v, [lax.broadcast(e, (LANES,)).astype(jnp.int32)])
                for c in range(D // LANES):
                    rows[e, pl.ds(c * LANES, LANES)] = (
                        rows[e, pl.ds(c * LANES, LANES)] * s)
                return _
            lax.fori_loop(0, K, scale, None, unroll=8)

            pltpu.sync_copy(rows, acc_sh.at[dst_v], add=True)
            return _
        lax.fori_loop(0, nch, body, None)

        plsc.subcore_barrier()

        def co(i, _):
            b = sid + i * NS
            pltpu.sync_copy(acc_sh.at[pl.ds(b * ZR, ZR)],
                            out_hbm.at[cid, pl.ds(b * ZR, ZR)])
            return _
        lax.fori_loop(0, nblk_s, co, None)

    return k(y, src, dst, ew)


def kernel(x, edge_index, edgenet_input, pae_w1, pae_b1, pae_w2, pae_b2,
           conv1_w, conv1_b, conv2_w, conv2_b, lin1_w, lin1_b,
           lin2_w, lin2_b):
    N, D = x.shape
    E = edge_index.shape[1]
    nclass = lin2_w.shape[1]

    src = edge_index[0]
    dst = edge_index[1]
    en_t = edgenet_input.T

    ew = _edge_mlp(en_t, pae_w1, pae_b1, pae_w2, pae_b2)
    xw1 = _matmul(x, conv1_w)

    degp = _sc_degree(dst, ew, N)
    degt = degp.T

    y1 = _prep(degt, xw1)
    acc1 = _sc_messages(y1, src, dst, ew)
    y2 = _mid(acc1, y1, degt, conv1_b, conv2_w)
    acc2 = _sc_messages(y2, src, dst, ew)

    l2wp = jnp.zeros((lin1_w.shape[1], D), jnp.float32)
    l2wp = l2wp.at[:, :nclass].set(lin2_w)
    l2bp = jnp.zeros((D,), jnp.float32).at[:nclass].set(lin2_b)
    out = _final(acc2, y2, degt, conv2_b, lin1_w, lin1_b, l2wp, l2bp)
    return out[:, :nclass]


# speedup vs baseline: 22.4129x; 2.1109x over previous
import functools

import jax
import jax.numpy as jnp
from jax import lax
from jax.experimental import pallas as pl
from jax.experimental.pallas import tpu as pltpu
from jax.experimental.pallas import tpu_sc as plsc

NC = 2
NS = 16
NW = NC * NS
LANES = 16
K = 128


def _mlp_body(en_ref, w1t_ref, b1c_ref, w2t_ref, b2_ref, out_ref):
    en = en_ref[0]
    h = jnp.dot(w1t_ref[...], en, preferred_element_type=jnp.float32)
    h = jnp.maximum(h + b1c_ref[...], 0.0)
    s = jnp.dot(w2t_ref[...], h, preferred_element_type=jnp.float32)
    out_ref[0] = jax.nn.sigmoid(s + b2_ref[...])


def _edge_mlp(en_t, w1, b1, w2, b2):
    E = en_t.shape[1]
    RB = 8000
    grid = E // RB
    en3 = en_t.reshape(2, grid, RB).swapaxes(0, 1)
    out = pl.pallas_call(
        _mlp_body,
        grid=(grid,),
        in_specs=[
            pl.BlockSpec((1, 2, RB), lambda i: (i, 0, 0)),
            pl.BlockSpec((64, 2), lambda i: (0, 0)),
            pl.BlockSpec((64, 1), lambda i: (0, 0)),
            pl.BlockSpec((1, 64), lambda i: (0, 0)),
            pl.BlockSpec((1, 1), lambda i: (0, 0)),
        ],
        out_specs=pl.BlockSpec((1, 1, RB), lambda i: (i, 0, 0)),
        out_shape=jax.ShapeDtypeStruct((grid, 1, RB), jnp.float32),
    )(en3, w1.T, b1[:, None], w2.T, b2[None, :])
    return out.reshape(E)


def _mm_body(x_ref, w_ref, o_ref):
    o_ref[...] = jnp.dot(x_ref[...], w_ref[...],
                         preferred_element_type=jnp.float32)


def _matmul(x, w):
    N, Kd = x.shape
    M = w.shape[1]
    R = 2000
    return pl.pallas_call(
        _mm_body,
        grid=(N // R,),
        in_specs=[
            pl.BlockSpec((R, Kd), lambda i: (i, 0)),
            pl.BlockSpec((Kd, M), lambda i: (0, 0)),
        ],
        out_specs=pl.BlockSpec((R, M), lambda i: (i, 0)),
        out_shape=jax.ShapeDtypeStruct((N, M), jnp.float32),
    )(x, w)


def _dinv_of(p):
    return lax.rsqrt(1.0 + p[:, 0:1] + p[:, 1:2])


def _prep_body(degt_ref, xw_ref, y_ref):
    y_ref[...] = xw_ref[...] * _dinv_of(degt_ref[...])


def _prep(degt, xw):
    N, D = xw.shape
    R = 2000
    return pl.pallas_call(
        _prep_body,
        grid=(N // R,),
        in_specs=[
            pl.BlockSpec((R, 2), lambda i: (i, 0)),
            pl.BlockSpec((R, D), lambda i: (i, 0)),
        ],
        out_specs=pl.BlockSpec((R, D), lambda i: (i, 0)),
        out_shape=jax.ShapeDtypeStruct((N, D), jnp.float32),
    )(degt, xw)


def _mid_body(acc_ref, y_ref, degt_ref, b_ref, w2_ref, y2_ref):
    dinv = _dinv_of(degt_ref[...])
    t = (acc_ref[0] + acc_ref[1] + y_ref[...]) * dinv + b_ref[...]
    h1 = jnp.maximum(t, 0.0)
    y2_ref[...] = jnp.dot(h1, w2_ref[...],
                          preferred_element_type=jnp.float32) * dinv


def _mid(acc, y, degt, b, w2):
    N, D = y.shape
    R = 2000
    return pl.pallas_call(
        _mid_body,
        grid=(N // R,),
        in_specs=[
            pl.BlockSpec((2, R, D), lambda i: (0, i, 0)),
            pl.BlockSpec((R, D), lambda i: (i, 0)),
            pl.BlockSpec((R, 2), lambda i: (i, 0)),
            pl.BlockSpec((1, D), lambda i: (0, 0)),
            pl.BlockSpec((D, D), lambda i: (0, 0)),
        ],
        out_specs=pl.BlockSpec((R, D), lambda i: (i, 0)),
        out_shape=jax.ShapeDtypeStruct((N, D), jnp.float32),
    )(acc, y, degt, b[None, :], w2)


def _fin_body(acc_ref, y_ref, degt_ref, b_ref, l1w_ref, l1b_ref,
              l2w_ref, l2b_ref, o_ref):
    dinv = _dinv_of(degt_ref[...])
    t = (acc_ref[0] + acc_ref[1] + y_ref[...]) * dinv + b_ref[...]
    h2 = jnp.maximum(t, 0.0)
    h3 = jnp.dot(h2, l1w_ref[...], preferred_element_type=jnp.float32)
    h3 = jnp.maximum(h3 + l1b_ref[...], 0.0)
    o_ref[...] = jnp.dot(h3, l2w_ref[...],
                         preferred_element_type=jnp.float32) + l2b_ref[...]


def _final(acc, y, degt, b, l1w, l1b, l2wp, l2bp):
    N, D = y.shape
    H = l1w.shape[1]
    R = 2000
    return pl.pallas_call(
        _fin_body,
        grid=(N // R,),
        in_specs=[
            pl.BlockSpec((2, R, D), lambda i: (0, i, 0)),
            pl.BlockSpec((R, D), lambda i: (i, 0)),
            pl.BlockSpec((R, 2), lambda i: (i, 0)),
            pl.BlockSpec((1, D), lambda i: (0, 0)),
            pl.BlockSpec((D, H), lambda i: (0, 0)),
            pl.BlockSpec((1, H), lambda i: (0, 0)),
            pl.BlockSpec((H, D), lambda i: (0, 0)),
            pl.BlockSpec((1, D), lambda i: (0, 0)),
        ],
        out_specs=pl.BlockSpec((R, D), lambda i: (i, 0)),
        out_shape=jax.ShapeDtypeStruct((N, D), jnp.float32),
    )(acc, y, degt, b[None, :], l1w, l1b[None, :], l2wp, l2bp[None, :])


def _sc_degree(dst2, ew_p, N):
    NCH = dst2.shape[0]
    RW = NCH // NW
    ZC = 2000

    mesh = plsc.VectorSubcoreMesh(core_axis_name="c", subcore_axis_name="s")

    @functools.partial(
        pl.kernel,
        out_type=jax.ShapeDtypeStruct((NC, N), jnp.float32),
        mesh=mesh,
        compiler_params=pltpu.CompilerParams(needs_layout_passes=False),
        scratch_types=[
            pltpu.VMEM((RW, K), jnp.int32),
            pltpu.VMEM((RW * K,), jnp.float32),
            pltpu.VMEM((ZC,), jnp.float32),
            pltpu.SemaphoreType.DMA,
            pltpu.SemaphoreType.DMA,
            pltpu.VMEM_SHARED((N,), jnp.float32),
        ],
    )
    def k(dst_hbm, ew_hbm, out_hbm, dst_all, ew_all, zbuf, isem, ssem,
          deg_sh):
        cid = lax.axis_index("c")
        sid = lax.axis_index("s")
        wid = sid * NC + cid

        c1 = pltpu.async_copy(dst_hbm.at[pl.ds(wid * RW, RW)], dst_all, isem)
        c2 = pltpu.async_copy(ew_hbm.at[pl.ds(wid * RW * K, RW * K)],
                              ew_all, isem)

        def zb(i, _):
            zbuf[pl.ds(i * LANES, LANES)] = jnp.zeros((LANES,), jnp.float32)
            return _
        lax.fori_loop(0, ZC // LANES, zb, None)

        @pl.when(sid == 0)
        def _():
            for i in range(N // ZC):
                pltpu.sync_copy(zbuf, deg_sh.at[pl.ds(i * ZC, ZC)])

        plsc.subcore_barrier()
        c1.wait()
        c2.wait()

        def fire(i, _):
            pltpu.async_copy(ew_all.at[pl.ds(i * K, K)],
                             deg_sh.at[dst_all.at[i]], ssem, add=True)
            return _
        lax.fori_loop(0, RW, fire, None)

        def drain(i, _):
            pltpu.make_async_copy(ew_all.at[pl.ds(i * K, K)],
                                  deg_sh.at[dst_all.at[i]], ssem).wait()
            return _
        lax.fori_loop(0, RW, drain, None)

        plsc.subcore_barrier()

        @pl.when(sid == 0)
        def _():
            pltpu.sync_copy(deg_sh, out_hbm.at[cid])

    return k(dst2, ew_p)


def _sc_messages(y, src_p, dst2, ew_p):
    N, D = y.shape
    NCH = dst2.shape[0]
    RW = NCH // NW
    ZR = 80
    NBLK = N // ZR

    mesh = plsc.VectorSubcoreMesh(core_axis_name="c", subcore_axis_name="s")

    @functools.partial(
        pl.kernel,
        out_type=jax.ShapeDtypeStruct((NC, N, D), jnp.float32),
        mesh=mesh,
        compiler_params=pltpu.CompilerParams(needs_layout_passes=False),
        scratch_types=[
            pltpu.VMEM((RW, K), jnp.int32),
            pltpu.VMEM((K,), jnp.int32),
            pltpu.VMEM((K,), jnp.int32),
            pltpu.VMEM((K,), jnp.float32),
            pltpu.VMEM((K,), jnp.float32),
            pltpu.VMEM((2, K, D), jnp.float32),
            pltpu.SemaphoreType.DMA,
            pltpu.SemaphoreType.DMA,
            pltpu.SemaphoreType.DMA,
            pltpu.SemaphoreType.DMA,
            pltpu.SemaphoreType.DMA,
            pltpu.SemaphoreType.DMA,
            pltpu.SemaphoreType.DMA,
            pltpu.VMEM_SHARED((N, D), jnp.float32),
        ],
    )
    def k(y_hbm, src_hbm, dst_hbm, ew_hbm, out_hbm,
          dst_all, srcb0, srcb1, ewb0, ewb1, rows,
          isem, ss0, ss1, es0, es1, gs0, gs1, acc_sh):
        cid = lax.axis_index("c")
        sid = lax.axis_index("s")
        wid = sid * NC + cid
        srcbufs = (srcb0, srcb1)
        ewbufs = (ewb0, ewb1)
        ssems = (ss0, ss1)
        esems = (es0, es1)
        gsems = (gs0, gs1)
        base_e = wid * RW * K

        ci = pltpu.async_copy(dst_hbm.at[pl.ds(wid * RW, RW)], dst_all, isem)

        def sissue(i, b):
            pltpu.async_copy(src_hbm.at[pl.ds(base_e + i * K, K)],
                             srcbufs[b], ssems[b])

        def swait(i, b):
            pltpu.make_async_copy(src_hbm.at[pl.ds(base_e + i * K, K)],
                                  srcbufs[b], ssems[b]).wait()

        def eissue(i, b):
            pltpu.async_copy(ew_hbm.at[pl.ds(base_e + i * K, K)],
                             ewbufs[b], esems[b])

        def ewait(i, b):
            pltpu.make_async_copy(ew_hbm.at[pl.ds(base_e + i * K, K)],
                                  ewbufs[b], esems[b]).wait()

        def gissue(i, b):
            pltpu.async_copy(y_hbm.at[srcbufs[b]], rows.at[b], gsems[b])

        def gwait(i, b):
            pltpu.make_async_copy(y_hbm.at[srcbufs[b]], rows.at[b],
                                  gsems[b]).wait()

        sissue(0, 0)
        sissue(1, 1)
        eissue(0, 0)
        eissue(1, 1)

        def zb(r, _):
            for c in range(D // LANES):
                rows[0, r, pl.ds(c * LANES, LANES)] = jnp.zeros(
                    (LANES,), jnp.float32)
            return _
        lax.fori_loop(0, ZR, zb, None)

        nblk_s = NBLK // NS + jnp.where(sid < NBLK % NS, 1, 0)

        def zc(i, _):
            blk = sid + i * NS
            pltpu.sync_copy(rows.at[0, pl.ds(0, ZR)],
                            acc_sh.at[pl.ds(blk * ZR, ZR)])
            return _
        lax.fori_loop(0, nblk_s, zc, None)

        plsc.subcore_barrier()
        ci.wait()
        swait(0, 0)
        gissue(0, 0)

        def outer(o, _):
            for b in range(2):
                i = o * 2 + b

                @pl.when(i + 1 < RW)
                def _():
                    swait(i + 1, 1 - b)
                    gissue(i + 1, 1 - b)

                gwait(i, b)

                @pl.when(i + 2 < RW)
                def _():
                    sissue(i + 2, b)

                ewait(i, b)

                def scale(e, _):
                    s = plsc.load_gather(
                        ewbufs[b],
                        [lax.broadcast(e, (LANES,)).astype(jnp.int32)])
                    for c in range(D // LANES):
                        rows[b, e, pl.ds(c * LANES, LANES)] = (
                            rows[b, e, pl.ds(c * LANES, LANES)] * s)
                    return _
                lax.fori_loop(0, K, scale, None, unroll=8)

                pltpu.sync_copy(rows.at[b], acc_sh.at[dst_all.at[i]],
                                add=True)

                @pl.when(i + 2 < RW)
                def _():
                    eissue(i + 2, b)
            return _
        lax.fori_loop(0, RW // 2, outer, None)

        plsc.subcore_barrier()

        def co(i, _):
            blk = sid + i * NS
            pltpu.sync_copy(acc_sh.at[pl.ds(blk * ZR, ZR)],
                            out_hbm.at[cid, pl.ds(blk * ZR, ZR)])
            return _
        lax.fori_loop(0, nblk_s, co, None)

    return k(y, src_p, dst2, ew_p)


def kernel(x, edge_index, edgenet_input, pae_w1, pae_b1, pae_w2, pae_b2,
           conv1_w, conv1_b, conv2_w, conv2_b, lin1_w, lin1_b,
           lin2_w, lin2_b):
    N, D = x.shape
    E = edge_index.shape[1]
    nclass = lin2_w.shape[1]

    src = edge_index[0]
    dst = edge_index[1]
    en_t = edgenet_input.T

    ew = _edge_mlp(en_t, pae_w1, pae_b1, pae_w2, pae_b2)
    xw1 = _matmul(x, conv1_w)

    ch2 = 2 * K * NW
    ep = ((E + ch2 - 1) // ch2) * ch2
    pad = ep - E
    padidx = jnp.arange(pad, dtype=jnp.int32) % N
    src_p = jnp.concatenate([src, padidx])
    dst2 = jnp.concatenate([dst, padidx]).reshape(-1, K)
    ew_p = jnp.concatenate([ew, jnp.zeros((pad,), jnp.float32)])

    degp = _sc_degree(dst2, ew_p, N)
    degt = degp.T

    y1 = _prep(degt, xw1)
    acc1 = _sc_messages(y1, src_p, dst2, ew_p)
    y2 = _mid(acc1, y1, degt, conv1_b, conv2_w)
    acc2 = _sc_messages(y2, src_p, dst2, ew_p)

    l2wp = jnp.zeros((lin1_w.shape[1], D), jnp.float32)
    l2wp = l2wp.at[:, :nclass].set(lin2_w)
    l2bp = jnp.zeros((D,), jnp.float32).at[:nclass].set(lin2_b)
    out = _final(acc2, y2, degt, conv2_b, lin1_w, lin1_b, l2wp, l2bp)
    return out[:, :nclass]


# speedup vs baseline: 25.6941x; 1.1464x over previous
import functools

import jax
import jax.numpy as jnp
from jax import lax
from jax.experimental import pallas as pl
from jax.experimental.pallas import tpu as pltpu
from jax.experimental.pallas import tpu_sc as plsc

NC = 2
NS = 16
NW = NC * NS
LANES = 16
K = 128


def _mlp_body(en_ref, w1t_ref, b1c_ref, w2t_ref, b2_ref, out_ref):
    en = en_ref[0]
    h = jnp.dot(w1t_ref[...], en, preferred_element_type=jnp.float32)
    h = jnp.maximum(h + b1c_ref[...], 0.0)
    s = jnp.dot(w2t_ref[...], h, preferred_element_type=jnp.float32)
    out_ref[0] = jax.nn.sigmoid(s + b2_ref[...])


def _edge_mlp(en_t, w1, b1, w2, b2):
    E = en_t.shape[1]
    RB = 8000
    grid = E // RB
    en3 = en_t.reshape(2, grid, RB).swapaxes(0, 1)
    out = pl.pallas_call(
        _mlp_body,
        grid=(grid,),
        in_specs=[
            pl.BlockSpec((1, 2, RB), lambda i: (i, 0, 0)),
            pl.BlockSpec((64, 2), lambda i: (0, 0)),
            pl.BlockSpec((64, 1), lambda i: (0, 0)),
            pl.BlockSpec((1, 64), lambda i: (0, 0)),
            pl.BlockSpec((1, 1), lambda i: (0, 0)),
        ],
        out_specs=pl.BlockSpec((1, 1, RB), lambda i: (i, 0, 0)),
        out_shape=jax.ShapeDtypeStruct((grid, 1, RB), jnp.float32),
    )(en3, w1.T, b1[:, None], w2.T, b2[None, :])
    return out.reshape(E)


def _mm_body(x_ref, w_ref, o_ref):
    o_ref[...] = jnp.dot(x_ref[...], w_ref[...],
                         preferred_element_type=jnp.float32)


def _matmul(x, w):
    N, Kd = x.shape
    M = w.shape[1]
    R = 2000
    return pl.pallas_call(
        _mm_body,
        grid=(N // R,),
        in_specs=[
            pl.BlockSpec((R, Kd), lambda i: (i, 0)),
            pl.BlockSpec((Kd, M), lambda i: (0, 0)),
        ],
        out_specs=pl.BlockSpec((R, M), lambda i: (i, 0)),
        out_shape=jax.ShapeDtypeStruct((N, M), jnp.float32),
    )(x, w)


def _dinv_of(p):
    return lax.rsqrt(1.0 + p[:, 0:1] + p[:, 1:2])


def _prep_body(degt_ref, xw_ref, y_ref):
    y_ref[...] = xw_ref[...] * _dinv_of(degt_ref[...])


def _prep(degt, xw):
    N, D = xw.shape
    R = 2000
    return pl.pallas_call(
        _prep_body,
        grid=(N // R,),
        in_specs=[
            pl.BlockSpec((R, 2), lambda i: (i, 0)),
            pl.BlockSpec((R, D), lambda i: (i, 0)),
        ],
        out_specs=pl.BlockSpec((R, D), lambda i: (i, 0)),
        out_shape=jax.ShapeDtypeStruct((N, D), jnp.float32),
    )(degt, xw)


def _mid_body(acc_ref, y_ref, degt_ref, b_ref, w2_ref, y2_ref):
    dinv = _dinv_of(degt_ref[...])
    t = (acc_ref[0] + acc_ref[1] + y_ref[...]) * dinv + b_ref[...]
    h1 = jnp.maximum(t, 0.0)
    y2_ref[...] = jnp.dot(h1, w2_ref[...],
                          preferred_element_type=jnp.float32) * dinv


def _mid(acc, y, degt, b, w2):
    N, D = y.shape
    R = 2000
    return pl.pallas_call(
        _mid_body,
        grid=(N // R,),
        in_specs=[
            pl.BlockSpec((2, R, D), lambda i: (0, i, 0)),
            pl.BlockSpec((R, D), lambda i: (i, 0)),
            pl.BlockSpec((R, 2), lambda i: (i, 0)),
            pl.BlockSpec((1, D), lambda i: (0, 0)),
            pl.BlockSpec((D, D), lambda i: (0, 0)),
        ],
        out_specs=pl.BlockSpec((R, D), lambda i: (i, 0)),
        out_shape=jax.ShapeDtypeStruct((N, D), jnp.float32),
    )(acc, y, degt, b[None, :], w2)


def _fin_body(acc_ref, y_ref, degt_ref, b_ref, l1w_ref, l1b_ref,
              l2w_ref, l2b_ref, o_ref):
    dinv = _dinv_of(degt_ref[...])
    t = (acc_ref[0] + acc_ref[1] + y_ref[...]) * dinv + b_ref[...]
    h2 = jnp.maximum(t, 0.0)
    h3 = jnp.dot(h2, l1w_ref[...], preferred_element_type=jnp.float32)
    h3 = jnp.maximum(h3 + l1b_ref[...], 0.0)
    o_ref[...] = jnp.dot(h3, l2w_ref[...],
                         preferred_element_type=jnp.float32) + l2b_ref[...]


def _final(acc, y, degt, b, l1w, l1b, l2wp, l2bp):
    N, D = y.shape
    H = l1w.shape[1]
    R = 2000
    return pl.pallas_call(
        _fin_body,
        grid=(N // R,),
        in_specs=[
            pl.BlockSpec((2, R, D), lambda i: (0, i, 0)),
            pl.BlockSpec((R, D), lambda i: (i, 0)),
            pl.BlockSpec((R, 2), lambda i: (i, 0)),
            pl.BlockSpec((1, D), lambda i: (0, 0)),
            pl.BlockSpec((D, H), lambda i: (0, 0)),
            pl.BlockSpec((1, H), lambda i: (0, 0)),
            pl.BlockSpec((H, D), lambda i: (0, 0)),
            pl.BlockSpec((1, D), lambda i: (0, 0)),
        ],
        out_specs=pl.BlockSpec((R, D), lambda i: (i, 0)),
        out_shape=jax.ShapeDtypeStruct((N, D), jnp.float32),
    )(acc, y, degt, b[None, :], l1w, l1b[None, :], l2wp, l2bp[None, :])


def _sc_degree(dst_p, ew_p, N):
    NCH = dst_p.shape[0] // K
    RW = NCH // NW
    ZC = 2000

    mesh = plsc.VectorSubcoreMesh(core_axis_name="c", subcore_axis_name="s")

    @functools.partial(
        pl.kernel,
        out_type=jax.ShapeDtypeStruct((NC, N), jnp.float32),
        mesh=mesh,
        compiler_params=pltpu.CompilerParams(needs_layout_passes=False),
        scratch_types=[
            pltpu.VMEM((RW, K), jnp.int32),
            pltpu.VMEM((RW * K,), jnp.float32),
            pltpu.VMEM((ZC,), jnp.float32),
            pltpu.SemaphoreType.DMA,
            pltpu.SemaphoreType.DMA,
            pltpu.VMEM_SHARED((N,), jnp.float32),
        ],
    )
    def k(dst_hbm, ew_hbm, out_hbm, dst_all, ew_all, zbuf, isem, ssem,
          deg_sh):
        cid = lax.axis_index("c")
        sid = lax.axis_index("s")
        wid = sid * NC + cid
        base_e = wid * RW * K

        def ifire(i, _):
            pltpu.async_copy(dst_hbm.at[pl.ds(base_e + i * K, K)],
                             dst_all.at[i], isem)
            return _
        lax.fori_loop(0, RW, ifire, None)
        c2 = pltpu.async_copy(ew_hbm.at[pl.ds(base_e, RW * K)],
                              ew_all, isem)

        def zb(i, _):
            zbuf[pl.ds(i * LANES, LANES)] = jnp.zeros((LANES,), jnp.float32)
            return _
        lax.fori_loop(0, ZC // LANES, zb, None)

        @pl.when(sid == 0)
        def _():
            for i in range(N // ZC):
                pltpu.sync_copy(zbuf, deg_sh.at[pl.ds(i * ZC, ZC)])

        plsc.subcore_barrier()

        def idrain(i, _):
            pltpu.make_async_copy(dst_hbm.at[pl.ds(base_e + i * K, K)],
                                  dst_all.at[i], isem).wait()
            return _
        lax.fori_loop(0, RW, idrain, None)
        c2.wait()

        def fire(i, _):
            pltpu.async_copy(ew_all.at[pl.ds(i * K, K)],
                             deg_sh.at[dst_all.at[i]], ssem, add=True)
            return _
        lax.fori_loop(0, RW, fire, None)

        def drain(i, _):
            pltpu.make_async_copy(ew_all.at[pl.ds(i * K, K)],
                                  deg_sh.at[dst_all.at[i]], ssem).wait()
            return _
        lax.fori_loop(0, RW, drain, None)

        plsc.subcore_barrier()

        @pl.when(sid == 0)
        def _():
            pltpu.sync_copy(deg_sh, out_hbm.at[cid])

    return k(dst_p, ew_p)


def _sc_messages(y, src_p, dst_p, ew_p):
    N, D = y.shape
    NCH = dst_p.shape[0] // K
    RW = NCH // NW
    NBUF = 3
    ZR = 80
    NBLK = N // ZR

    mesh = plsc.VectorSubcoreMesh(core_axis_name="c", subcore_axis_name="s")

    @functools.partial(
        pl.kernel,
        out_type=jax.ShapeDtypeStruct((NC, N, D), jnp.float32),
        mesh=mesh,
        compiler_params=pltpu.CompilerParams(needs_layout_passes=False),
        scratch_types=(
            [pltpu.VMEM((K,), jnp.int32) for _ in range(NBUF)]
            + [pltpu.VMEM((K,), jnp.int32) for _ in range(NBUF)]
            + [pltpu.VMEM((K,), jnp.float32) for _ in range(NBUF)]
            + [pltpu.VMEM((NBUF, K, D), jnp.float32)]
            + [pltpu.SemaphoreType.DMA for _ in range(5 * NBUF)]
            + [pltpu.VMEM_SHARED((N, D), jnp.float32)]
        ),
    )
    def k(y_hbm, src_hbm, dst_hbm, ew_hbm, out_hbm, *refs):
        srcbufs = refs[0:NBUF]
        dstbufs = refs[NBUF:2 * NBUF]
        ewbufs = refs[2 * NBUF:3 * NBUF]
        rows = refs[3 * NBUF]
        ssems = refs[3 * NBUF + 1:4 * NBUF + 1]
        dsems = refs[4 * NBUF + 1:5 * NBUF + 1]
        esems = refs[5 * NBUF + 1:6 * NBUF + 1]
        gsems = refs[6 * NBUF + 1:7 * NBUF + 1]
        csems = refs[7 * NBUF + 1:8 * NBUF + 1]
        acc_sh = refs[8 * NBUF + 1]

        cid = lax.axis_index("c")
        sid = lax.axis_index("s")
        wid = sid * NC + cid
        base_e = wid * RW * K

        def sissue(i, b):
            pltpu.async_copy(src_hbm.at[pl.ds(base_e + i * K, K)],
                             srcbufs[b], ssems[b])

        def swait(i, b):
            pltpu.make_async_copy(src_hbm.at[pl.ds(base_e + i * K, K)],
                                  srcbufs[b], ssems[b]).wait()

        def dissue(i, b):
            pltpu.async_copy(dst_hbm.at[pl.ds(base_e + i * K, K)],
                             dstbufs[b], dsems[b])

        def dwait(i, b):
            pltpu.make_async_copy(dst_hbm.at[pl.ds(base_e + i * K, K)],
                                  dstbufs[b], dsems[b]).wait()

        def eissue(i, b):
            pltpu.async_copy(ew_hbm.at[pl.ds(base_e + i * K, K)],
                             ewbufs[b], esems[b])

        def ewait(i, b):
            pltpu.make_async_copy(ew_hbm.at[pl.ds(base_e + i * K, K)],
                                  ewbufs[b], esems[b]).wait()

        def gissue(i, b):
            pltpu.async_copy(y_hbm.at[srcbufs[b]], rows.at[b], gsems[b])

        def gwait(i, b):
            pltpu.make_async_copy(y_hbm.at[srcbufs[b]], rows.at[b],
                                  gsems[b]).wait()

        def scissue(i, b):
            pltpu.async_copy(rows.at[b], acc_sh.at[dstbufs[b]], csems[b],
                             add=True)

        def scwait(i, b):
            pltpu.make_async_copy(rows.at[b], acc_sh.at[dstbufs[b]],
                                  csems[b]).wait()

        for j in range(2):
            sissue(j, j)
        for j in range(NBUF):
            dissue(j, j)
            eissue(j, j)

        def zb(r, _):
            for c in range(D // LANES):
                rows[2, r, pl.ds(c * LANES, LANES)] = jnp.zeros(
                    (LANES,), jnp.float32)
            return _
        lax.fori_loop(0, ZR, zb, None)

        nblk_s = NBLK // NS + jnp.where(sid < NBLK % NS, 1, 0)

        def zc(i, _):
            blk = sid + i * NS
            pltpu.sync_copy(rows.at[2, pl.ds(0, ZR)],
                            acc_sh.at[pl.ds(blk * ZR, ZR)])
            return _
        lax.fori_loop(0, nblk_s, zc, None)

        plsc.subcore_barrier()
        swait(0, 0)
        gissue(0, 0)

        def outer(o, _):
            for b in range(NBUF):
                i = o * NBUF + b

                @pl.when(i >= 2)
                def _():
                    scwait(i - 2, (b + 1) % NBUF)

                @pl.when(i + 1 < RW)
                def _():
                    swait(i + 1, (b + 1) % NBUF)
                    gissue(i + 1, (b + 1) % NBUF)

                @pl.when((i + 1 >= NBUF) & (i + 1 < RW))
                def _():
                    dissue(i + 1, (b + 1) % NBUF)

                gwait(i, b)

                @pl.when(i + 2 < RW)
                def _():
                    sissue(i + 2, (b + 2) % NBUF)

                ewait(i, b)

                def scale(e, _):
                    s = plsc.load_gather(
                        ewbufs[b],
                        [lax.broadcast(e, (LANES,)).astype(jnp.int32)])
                    for c in range(D // LANES):
                        rows[b, e, pl.ds(c * LANES, LANES)] = (
                            rows[b, e, pl.ds(c * LANES, LANES)] * s)
                    return _
                lax.fori_loop(0, K, scale, None, unroll=8)

                dwait(i, b)
                scissue(i, b)

                @pl.when(i + NBUF < RW)
                def _():
                    eissue(i + NBUF, b)
            return _
        lax.fori_loop(0, RW // NBUF, outer, None)

        scwait(RW - 2, (RW - 2) % NBUF)
        scwait(RW - 1, (RW - 1) % NBUF)

        plsc.subcore_barrier()

        def co(i, _):
            blk = sid + i * NS
            pltpu.sync_copy(acc_sh.at[pl.ds(blk * ZR, ZR)],
                            out_hbm.at[cid, pl.ds(blk * ZR, ZR)])
            return _
        lax.fori_loop(0, nblk_s, co, None)

    return k(y, src_p, dst_p, ew_p)


def kernel(x, edge_index, edgenet_input, pae_w1, pae_b1, pae_w2, pae_b2,
           conv1_w, conv1_b, conv2_w, conv2_b, lin1_w, lin1_b,
           lin2_w, lin2_b):
    N, D = x.shape
    E = edge_index.shape[1]
    nclass = lin2_w.shape[1]

    src = edge_index[0]
    dst = edge_index[1]
    en_t = edgenet_input.T

    ew = _edge_mlp(en_t, pae_w1, pae_b1, pae_w2, pae_b2)
    xw1 = _matmul(x, conv1_w)

    ch2 = 3 * K * NW
    ep = ((E + ch2 - 1) // ch2) * ch2
    pad = ep - E
    padidx = jnp.arange(pad, dtype=jnp.int32) % N
    src_p = jnp.concatenate([src, padidx])
    dst_p = jnp.concatenate([dst, padidx])
    ew_p = jnp.concatenate([ew, jnp.zeros((pad,), jnp.float32)])

    degp = _sc_degree(dst_p, ew_p, N)
    degt = degp.T

    y1 = _prep(degt, xw1)
    acc1 = _sc_messages(y1, src_p, dst_p, ew_p)
    y2 = _mid(acc1, y1, degt, conv1_b, conv2_w)
    acc2 = _sc_messages(y2, src_p, dst_p, ew_p)

    l2wp = jnp.zeros((lin1_w.shape[1], D), jnp.float32)
    l2wp = l2wp.at[:, :nclass].set(lin2_w)
    l2bp = jnp.zeros((D,), jnp.float32).at[:nclass].set(lin2_b)
    out = _final(acc2, y2, degt, conv2_b, lin1_w, lin1_b, l2wp, l2bp)
    return out[:, :nclass]


# speedup vs baseline: 27.3709x; 1.0653x over previous
import functools

import jax
import jax.numpy as jnp
from jax import lax
from jax.experimental import pallas as pl
from jax.experimental.pallas import tpu as pltpu
from jax.experimental.pallas import tpu_sc as plsc

NC = 2
NS = 16
NW = NC * NS
LANES = 16
K = 128


def _mlp_body(en_ref, w1t_ref, b1c_ref, w2t_ref, b2_ref, out_ref):
    en = en_ref[0]
    h = jnp.dot(w1t_ref[...], en, preferred_element_type=jnp.float32)
    h = jnp.maximum(h + b1c_ref[...], 0.0)
    s = jnp.dot(w2t_ref[...], h, preferred_element_type=jnp.float32)
    out_ref[0] = jax.nn.sigmoid(s + b2_ref[...])


def _edge_mlp(en_t, w1, b1, w2, b2):
    E = en_t.shape[1]
    RB = 8000
    grid = E // RB
    en3 = en_t.reshape(2, grid, RB).swapaxes(0, 1)
    out = pl.pallas_call(
        _mlp_body,
        grid=(grid,),
        in_specs=[
            pl.BlockSpec((1, 2, RB), lambda i: (i, 0, 0)),
            pl.BlockSpec((64, 2), lambda i: (0, 0)),
            pl.BlockSpec((64, 1), lambda i: (0, 0)),
            pl.BlockSpec((1, 64), lambda i: (0, 0)),
            pl.BlockSpec((1, 1), lambda i: (0, 0)),
        ],
        out_specs=pl.BlockSpec((1, 1, RB), lambda i: (i, 0, 0)),
        out_shape=jax.ShapeDtypeStruct((grid, 1, RB), jnp.float32),
    )(en3, w1.T, b1[:, None], w2.T, b2[None, :])
    return out.reshape(E)


def _dinv_of(p):
    return lax.rsqrt(1.0 + p[:, 0:1] + p[:, 1:2])


def _prep_body(degt_ref, x_ref, w_ref, y_ref):
    xw = jnp.dot(x_ref[...], w_ref[...], preferred_element_type=jnp.float32)
    y_ref[...] = xw * _dinv_of(degt_ref[...])


def _prep(degt, x, w):
    N, D = x.shape
    M = w.shape[1]
    R = 2000
    return pl.pallas_call(
        _prep_body,
        grid=(N // R,),
        in_specs=[
            pl.BlockSpec((R, 2), lambda i: (i, 0)),
            pl.BlockSpec((R, D), lambda i: (i, 0)),
            pl.BlockSpec((D, M), lambda i: (0, 0)),
        ],
        out_specs=pl.BlockSpec((R, M), lambda i: (i, 0)),
        out_shape=jax.ShapeDtypeStruct((N, M), jnp.float32),
    )(degt, x, w)


def _mid_body(acc_ref, y_ref, degt_ref, b_ref, w2_ref, y2_ref):
    dinv = _dinv_of(degt_ref[...])
    t = (acc_ref[0] + acc_ref[1] + y_ref[...]) * dinv + b_ref[...]
    h1 = jnp.maximum(t, 0.0)
    y2_ref[...] = jnp.dot(h1, w2_ref[...],
                          preferred_element_type=jnp.float32) * dinv


def _mid(acc, y, degt, b, w2):
    N, D = y.shape
    R = 2000
    return pl.pallas_call(
        _mid_body,
        grid=(N // R,),
        in_specs=[
            pl.BlockSpec((2, R, D), lambda i: (0, i, 0)),
            pl.BlockSpec((R, D), lambda i: (i, 0)),
            pl.BlockSpec((R, 2), lambda i: (i, 0)),
            pl.BlockSpec((1, D), lambda i: (0, 0)),
            pl.BlockSpec((D, D), lambda i: (0, 0)),
        ],
        out_specs=pl.BlockSpec((R, D), lambda i: (i, 0)),
        out_shape=jax.ShapeDtypeStruct((N, D), jnp.float32),
    )(acc, y, degt, b[None, :], w2)


def _fin_body(acc_ref, y_ref, degt_ref, b_ref, l1w_ref, l1b_ref,
              l2w_ref, l2b_ref, o_ref):
    dinv = _dinv_of(degt_ref[...])
    t = (acc_ref[0] + acc_ref[1] + y_ref[...]) * dinv + b_ref[...]
    h2 = jnp.maximum(t, 0.0)
    h3 = jnp.dot(h2, l1w_ref[...], preferred_element_type=jnp.float32)
    h3 = jnp.maximum(h3 + l1b_ref[...], 0.0)
    o_ref[...] = jnp.dot(h3, l2w_ref[...],
                         preferred_element_type=jnp.float32) + l2b_ref[...]


def _final(acc, y, degt, b, l1w, l1b, l2wp, l2bp):
    N, D = y.shape
    H = l1w.shape[1]
    R = 2000
    return pl.pallas_call(
        _fin_body,
        grid=(N // R,),
        in_specs=[
            pl.BlockSpec((2, R, D), lambda i: (0, i, 0)),
            pl.BlockSpec((R, D), lambda i: (i, 0)),
            pl.BlockSpec((R, 2), lambda i: (i, 0)),
            pl.BlockSpec((1, D), lambda i: (0, 0)),
            pl.BlockSpec((D, H), lambda i: (0, 0)),
            pl.BlockSpec((1, H), lambda i: (0, 0)),
            pl.BlockSpec((H, D), lambda i: (0, 0)),
            pl.BlockSpec((1, D), lambda i: (0, 0)),
        ],
        out_specs=pl.BlockSpec((R, D), lambda i: (i, 0)),
        out_shape=jax.ShapeDtypeStruct((N, D), jnp.float32),
    )(acc, y, degt, b[None, :], l1w, l1b[None, :], l2wp, l2bp[None, :])


def _sc_degree(dst_p, ew_p, N):
    NCH = dst_p.shape[0] // K
    RW = NCH // NW
    ZC = 2000

    mesh = plsc.VectorSubcoreMesh(core_axis_name="c", subcore_axis_name="s")

    @functools.partial(
        pl.kernel,
        out_type=jax.ShapeDtypeStruct((NC, N), jnp.float32),
        mesh=mesh,
        compiler_params=pltpu.CompilerParams(needs_layout_passes=False),
        scratch_types=[
            pltpu.VMEM((RW, K), jnp.int32),
            pltpu.VMEM((RW * K,), jnp.float32),
            pltpu.VMEM((ZC,), jnp.float32),
            pltpu.SemaphoreType.DMA,
            pltpu.SemaphoreType.DMA,
            pltpu.VMEM_SHARED((N,), jnp.float32),
        ],
    )
    def k(dst_hbm, ew_hbm, out_hbm, dst_all, ew_all, zbuf, isem, ssem,
          deg_sh):
        cid = lax.axis_index("c")
        sid = lax.axis_index("s")
        wid = sid * NC + cid
        base_e = wid * RW * K

        def ifire(i, _):
            pltpu.async_copy(dst_hbm.at[pl.ds(base_e + i * K, K)],
                             dst_all.at[i], isem)
            return _
        lax.fori_loop(0, RW, ifire, None)
        c2 = pltpu.async_copy(ew_hbm.at[pl.ds(base_e, RW * K)],
                              ew_all, isem)

        def zb(i, _):
            zbuf[pl.ds(i * LANES, LANES)] = jnp.zeros((LANES,), jnp.float32)
            return _
        lax.fori_loop(0, ZC // LANES, zb, None)

        @pl.when(sid == 0)
        def _():
            for i in range(N // ZC):
                pltpu.sync_copy(zbuf, deg_sh.at[pl.ds(i * ZC, ZC)])

        plsc.subcore_barrier()

        def idrain(i, _):
            pltpu.make_async_copy(dst_hbm.at[pl.ds(base_e + i * K, K)],
                                  dst_all.at[i], isem).wait()
            return _
        lax.fori_loop(0, RW, idrain, None)
        c2.wait()

        def fire(i, _):
            pltpu.async_copy(ew_all.at[pl.ds(i * K, K)],
                             deg_sh.at[dst_all.at[i]], ssem, add=True)
            return _
        lax.fori_loop(0, RW, fire, None)

        def drain(i, _):
            pltpu.make_async_copy(ew_all.at[pl.ds(i * K, K)],
                                  deg_sh.at[dst_all.at[i]], ssem).wait()
            return _
        lax.fori_loop(0, RW, drain, None)

        plsc.subcore_barrier()

        @pl.when(sid == 0)
        def _():
            pltpu.sync_copy(deg_sh, out_hbm.at[cid])

    return k(dst_p, ew_p)


def _sc_messages(y, src_p, dst_p, ew_p):
    N, D = y.shape
    NCH = dst_p.shape[0] // K
    RW = NCH // NW
    NBUF = 3
    ZR = 80
    NBLK = N // ZR

    mesh = plsc.VectorSubcoreMesh(core_axis_name="c", subcore_axis_name="s")

    @functools.partial(
        pl.kernel,
        out_type=jax.ShapeDtypeStruct((NC, N, D), jnp.float32),
        mesh=mesh,
        compiler_params=pltpu.CompilerParams(needs_layout_passes=False),
        scratch_types=(
            [pltpu.VMEM((K,), jnp.int32) for _ in range(NBUF)]
            + [pltpu.VMEM((K,), jnp.int32) for _ in range(NBUF)]
            + [pltpu.VMEM((K,), jnp.float32) for _ in range(NBUF)]
            + [pltpu.VMEM((NBUF, K, D), jnp.float32)]
            + [pltpu.SemaphoreType.DMA for _ in range(5 * NBUF)]
            + [pltpu.VMEM_SHARED((N, D), jnp.float32)]
        ),
    )
    def k(y_hbm, src_hbm, dst_hbm, ew_hbm, out_hbm, *refs):
        srcbufs = refs[0:NBUF]
        dstbufs = refs[NBUF:2 * NBUF]
        ewbufs = refs[2 * NBUF:3 * NBUF]
        rows = refs[3 * NBUF]
        ssems = refs[3 * NBUF + 1:4 * NBUF + 1]
        dsems = refs[4 * NBUF + 1:5 * NBUF + 1]
        esems = refs[5 * NBUF + 1:6 * NBUF + 1]
        gsems = refs[6 * NBUF + 1:7 * NBUF + 1]
        csems = refs[7 * NBUF + 1:8 * NBUF + 1]
        acc_sh = refs[8 * NBUF + 1]

        cid = lax.axis_index("c")
        sid = lax.axis_index("s")
        wid = sid * NC + cid
        base_e = wid * RW * K

        def sissue(i, b):
            pltpu.async_copy(src_hbm.at[pl.ds(base_e + i * K, K)],
                             srcbufs[b], ssems[b])

        def swait(i, b):
            pltpu.make_async_copy(src_hbm.at[pl.ds(base_e + i * K, K)],
                                  srcbufs[b], ssems[b]).wait()

        def dissue(i, b):
            pltpu.async_copy(dst_hbm.at[pl.ds(base_e + i * K, K)],
                             dstbufs[b], dsems[b])

        def dwait(i, b):
            pltpu.make_async_copy(dst_hbm.at[pl.ds(base_e + i * K, K)],
                                  dstbufs[b], dsems[b]).wait()

        def eissue(i, b):
            pltpu.async_copy(ew_hbm.at[pl.ds(base_e + i * K, K)],
                             ewbufs[b], esems[b])

        def ewait(i, b):
            pltpu.make_async_copy(ew_hbm.at[pl.ds(base_e + i * K, K)],
                                  ewbufs[b], esems[b]).wait()

        def gissue(i, b):
            pltpu.async_copy(y_hbm.at[srcbufs[b]], rows.at[b], gsems[b])

        def gwait(i, b):
            pltpu.make_async_copy(y_hbm.at[srcbufs[b]], rows.at[b],
                                  gsems[b]).wait()

        def scissue(i, b):
            pltpu.async_copy(rows.at[b], acc_sh.at[dstbufs[b]], csems[b],
                             add=True)

        def scwait(i, b):
            pltpu.make_async_copy(rows.at[b], acc_sh.at[dstbufs[b]],
                                  csems[b]).wait()

        for j in range(2):
            sissue(j, j)
        for j in range(NBUF):
            dissue(j, j)
            eissue(j, j)

        def zb(r, _):
            for c in range(D // LANES):
                rows[2, r, pl.ds(c * LANES, LANES)] = jnp.zeros(
                    (LANES,), jnp.float32)
            return _
        lax.fori_loop(0, ZR, zb, None)

        nblk_s = NBLK // NS + jnp.where(sid < NBLK % NS, 1, 0)

        def zc(i, _):
            blk = sid + i * NS
            pltpu.sync_copy(rows.at[2, pl.ds(0, ZR)],
                            acc_sh.at[pl.ds(blk * ZR, ZR)])
            return _
        lax.fori_loop(0, nblk_s, zc, None)

        plsc.subcore_barrier()
        swait(0, 0)
        gissue(0, 0)

        def outer(o, _):
            for b in range(NBUF):
                i = o * NBUF + b

                @pl.when(i >= 2)
                def _():
                    scwait(i - 2, (b + 1) % NBUF)

                @pl.when(i + 1 < RW)
                def _():
                    swait(i + 1, (b + 1) % NBUF)
                    gissue(i + 1, (b + 1) % NBUF)

                @pl.when((i + 1 >= NBUF) & (i + 1 < RW))
                def _():
                    dissue(i + 1, (b + 1) % NBUF)

                gwait(i, b)

                @pl.when(i + 2 < RW)
                def _():
                    sissue(i + 2, (b + 2) % NBUF)

                ewait(i, b)

                def scale16(g, _):
                    ew16 = ewbufs[b][pl.ds(g * LANES, LANES)]
                    for j in range(LANES):
                        e = g * LANES + j
                        s = jnp.broadcast_to(ew16[j], (LANES,))
                        for c in range(D // LANES):
                            rows[b, e, pl.ds(c * LANES, LANES)] = (
                                rows[b, e, pl.ds(c * LANES, LANES)] * s)
                    return _
                lax.fori_loop(0, K // LANES, scale16, None)

                dwait(i, b)
                scissue(i, b)

                @pl.when(i + NBUF < RW)
                def _():
                    eissue(i + NBUF, b)
            return _
        lax.fori_loop(0, RW // NBUF, outer, None)

        scwait(RW - 2, (RW - 2) % NBUF)
        scwait(RW - 1, (RW - 1) % NBUF)

        plsc.subcore_barrier()

        def co(i, _):
            blk = sid + i * NS
            pltpu.sync_copy(acc_sh.at[pl.ds(blk * ZR, ZR)],
                            out_hbm.at[cid, pl.ds(blk * ZR, ZR)])
            return _
        lax.fori_loop(0, nblk_s, co, None)

    return k(y, src_p, dst_p, ew_p)


def kernel(x, edge_index, edgenet_input, pae_w1, pae_b1, pae_w2, pae_b2,
           conv1_w, conv1_b, conv2_w, conv2_b, lin1_w, lin1_b,
           lin2_w, lin2_b):
    N, D = x.shape
    E = edge_index.shape[1]
    nclass = lin2_w.shape[1]

    src = edge_index[0]
    dst = edge_index[1]
    en_t = edgenet_input.T

    ew = _edge_mlp(en_t, pae_w1, pae_b1, pae_w2, pae_b2)

    ch2 = 3 * K * NW
    ep = ((E + ch2 - 1) // ch2) * ch2
    pad = ep - E
    padidx = jnp.arange(pad, dtype=jnp.int32) % N
    src_p = jnp.concatenate([src, padidx])
    dst_p = jnp.concatenate([dst, padidx])
    ew_p = jnp.concatenate([ew, jnp.zeros((pad,), jnp.float32)])

    degp = _sc_degree(dst_p, ew_p, N)
    degt = degp.T

    y1 = _prep(degt, x, conv1_w)
    acc1 = _sc_messages(y1, src_p, dst_p, ew_p)
    y2 = _mid(acc1, y1, degt, conv1_b, conv2_w)
    acc2 = _sc_messages(y2, src_p, dst_p, ew_p)

    l2wp = jnp.zeros((lin1_w.shape[1], D), jnp.float32)
    l2wp = l2wp.at[:, :nclass].set(lin2_w)
    l2bp = jnp.zeros((D,), jnp.float32).at[:nclass].set(lin2_b)
    out = _final(acc2, y2, degt, conv2_b, lin1_w, lin1_b, l2wp, l2bp)
    return out[:, :nclass]
